# baseline (speedup 1.0000x reference)
import numpy as np
import jax
import jax.numpy as jnp
from jax.experimental import pallas as pl
from jax.experimental.pallas import tpu as pltpu

CB = 7
IMG = 256


def _randomize_betas_const(cb):
    rnd_lst = np.random.choice(2, cb, p=[0.5, 0.5]).tolist()
    for i in range(len(rnd_lst)):
        ln = len(rnd_lst) - 1
        if rnd_lst[i] == 0:
            rnd_lst[i] = rnd_lst[i] + i
        else:
            rnd_lst[i] = rnd_lst[i] + ln + i
    return rnd_lst


np.random.seed(0)
_SELS = [_randomize_betas_const(CB) for _ in range(4)]
_THETA_SEL = np.random.choice(2, 4, p=[0.5, 0.5]).tolist()

_RUNS = []
for _sel in _SELS:
    runs = []
    c0 = 0
    while c0 < CB:
        b2src = _sel[c0] >= CB
        c1 = c0
        while c1 < CB and (_sel[c1] >= CB) == b2src:
            c1 += 1
        runs.append((c0, c1 - c0, b2src))
        c0 = c1
    _RUNS.append(runs)
_N_COPIES = sum(len(r) for r in _RUNS) + 4


def _dma_kernel(b1, b2, t1c, t2c, o1, o2, o3, o4, tp1, tp2, sems):
    outs = (o1, o2, o3, o4)
    batch = tp1.shape[0]

    tp1[...] = jnp.broadcast_to(t1c[:, 0][None, :, None], (batch, IMG, IMG))
    tp2[...] = jnp.broadcast_to(t2c[:, 0][None, :, None], (batch, IMG, IMG))

    copies = []
    for k in range(4):
        for (c0, n, b2src) in _RUNS[k]:
            src = b2 if b2src else b1
            cp = pltpu.make_async_copy(
                src.at[:, pl.ds(c0, n)],
                outs[k].at[:, pl.ds(c0, n)],
                sems.at[len(copies)],
            )
            cp.start()
            copies.append(cp)
    for k in range(4):
        tp = tp2 if _THETA_SEL[k] else tp1
        cp = pltpu.make_async_copy(tp, outs[k].at[:, CB], sems.at[len(copies)])
        cp.start()
        copies.append(cp)

    for cp in copies:
        cp.wait()


def kernel(beta1, beta2, theta1, theta2):
    batch = beta1.shape[0]
    t1_col = jnp.tile(theta1[:, 0], IMG // batch).reshape(IMG, 1)
    t2_col = jnp.tile(theta2[:, 0], IMG // batch).reshape(IMG, 1)

    any_spec = pl.BlockSpec(memory_space=pl.ANY)
    vmem_spec = pl.BlockSpec(memory_space=pltpu.VMEM)
    out_shape = jax.ShapeDtypeStruct((batch, CB + 1, IMG, IMG), jnp.float32)

    outs = pl.pallas_call(
        _dma_kernel,
        in_specs=[any_spec, any_spec, vmem_spec, vmem_spec],
        out_specs=[any_spec] * 4,
        out_shape=[out_shape] * 4,
        scratch_shapes=[
            pltpu.VMEM((batch, IMG, IMG), jnp.float32),
            pltpu.VMEM((batch, IMG, IMG), jnp.float32),
            pltpu.SemaphoreType.DMA((_N_COPIES,)),
        ],
    )(beta1, beta2, t1_col, t2_col)

    return (*outs, jnp.array(_THETA_SEL, dtype=jnp.int32))


# speedup vs baseline: 42.4197x; 42.4197x over previous
import functools

import numpy as np
import jax
import jax.numpy as jnp
from jax import lax
from jax.experimental import pallas as pl
from jax.experimental.pallas import tpu as pltpu
from jax.experimental.pallas import tpu_sc as plsc

CB = 7
IMG = 256
BATCH = 32


def _randomize_betas_const(cb):
    rnd_lst = np.random.choice(2, cb, p=[0.5, 0.5]).tolist()
    for i in range(len(rnd_lst)):
        ln = len(rnd_lst) - 1
        if rnd_lst[i] == 0:
            rnd_lst[i] = rnd_lst[i] + i
        else:
            rnd_lst[i] = rnd_lst[i] + ln + i
    return rnd_lst


np.random.seed(0)
_SELS = [_randomize_betas_const(CB) for _ in range(4)]
_THETA_SEL = np.random.choice(2, 4, p=[0.5, 0.5]).tolist()

BB = 2

_SC_MESH = plsc.VectorSubcoreMesh(core_axis_name="c", subcore_axis_name="s")


@functools.partial(
    pl.kernel,
    out_type=jax.ShapeDtypeStruct((BATCH, CB + 1, IMG, IMG), jnp.float32),
    mesh=_SC_MESH,
    scratch_types=[pltpu.VMEM((IMG, IMG), jnp.float32)],
)
def _sc_out4(b1, b2, tplane, out, buf):
    b = lax.axis_index("s") * 2 + lax.axis_index("c")
    sel = _SELS[3]
    pltpu.sync_copy(tplane, buf)
    pltpu.sync_copy(buf, out.at[b, CB])
    for c in range(CB):
        src = b2 if sel[c] >= CB else b1
        pltpu.sync_copy(src.at[b, c], buf)
        pltpu.sync_copy(buf, out.at[b, c])


def _tc_kernel(b1_ref, b2_ref, t1_ref, t2_ref, o1_ref, o2_ref, o3_ref):
    outs = (o1_ref, o2_ref, o3_ref)
    for k in range(3):
        sel = _SELS[k]
        for c in range(CB):
            src = b2_ref if sel[c] >= CB else b1_ref
            outs[k][:, c] = src[:, c]
        t_ref = t2_ref if _THETA_SEL[k] else t1_ref
        tcol = t_ref[:, :]
        outs[k][:, CB] = jnp.broadcast_to(tcol[None, :, :], (BB, IMG, IMG))


def kernel(beta1, beta2, theta1, theta2):
    batch = beta1.shape[0]
    t1_col = jnp.tile(theta1[:, 0], IMG // batch).reshape(IMG, 1)
    t2_col = jnp.tile(theta2[:, 0], IMG // batch).reshape(IMG, 1)
    t4_plane = jnp.broadcast_to(
        t2_col if _THETA_SEL[3] else t1_col, (IMG, IMG)
    )

    out4 = _sc_out4(beta1, beta2, t4_plane)

    beta_spec = pl.BlockSpec((BB, CB, IMG, IMG), lambda b: (b, 0, 0, 0))
    theta_spec = pl.BlockSpec((IMG, 1), lambda b: (0, 0))
    out_spec = pl.BlockSpec((BB, CB + 1, IMG, IMG), lambda b: (b, 0, 0, 0))
    out_shape = jax.ShapeDtypeStruct((batch, CB + 1, IMG, IMG), jnp.float32)

    outs = pl.pallas_call(
        _tc_kernel,
        grid=(batch // BB,),
        in_specs=[beta_spec, beta_spec, theta_spec, theta_spec],
        out_specs=[out_spec] * 3,
        out_shape=[out_shape] * 3,
    )(beta1, beta2, t1_col, t2_col)

    return (*outs, out4, jnp.array(_THETA_SEL, dtype=jnp.int32))


# speedup vs baseline: 42.5013x; 1.0019x over previous
import functools

import numpy as np
import jax
import jax.numpy as jnp
from jax import lax
from jax.experimental import pallas as pl
from jax.experimental.pallas import tpu as pltpu
from jax.experimental.pallas import tpu_sc as plsc

CB = 7
IMG = 256
BATCH = 32


def _randomize_betas_const(cb):
    rnd_lst = np.random.choice(2, cb, p=[0.5, 0.5]).tolist()
    for i in range(len(rnd_lst)):
        ln = len(rnd_lst) - 1
        if rnd_lst[i] == 0:
            rnd_lst[i] = rnd_lst[i] + i
        else:
            rnd_lst[i] = rnd_lst[i] + ln + i
    return rnd_lst


np.random.seed(0)
_SELS = [_randomize_betas_const(CB) for _ in range(4)]
_THETA_SEL = np.random.choice(2, 4, p=[0.5, 0.5]).tolist()

BB = 2

_SC_MESH = plsc.VectorSubcoreMesh(core_axis_name="c", subcore_axis_name="s")


@functools.partial(
    pl.kernel,
    out_type=jax.ShapeDtypeStruct((BATCH, CB + 1, IMG, IMG), jnp.float32),
    mesh=_SC_MESH,
    scratch_types=[
        pltpu.VMEM((IMG // 2, IMG), jnp.float32),
        pltpu.VMEM((IMG // 2, IMG), jnp.float32),
        pltpu.SemaphoreType.DMA,
        pltpu.SemaphoreType.DMA,
    ],
)
def _sc_out4(b1, b2, tplane, out, bufa, bufb, sema, semb):
    b = lax.axis_index("s") * 2 + lax.axis_index("c")
    sel = _SELS[3]
    H = IMG // 2

    chunks = []
    for c in range(CB):
        src = b2 if sel[c] >= CB else b1
        for h in range(2):
            chunks.append((src.at[b, c, pl.ds(h * H, H)], out.at[b, c, pl.ds(h * H, H)]))
    for h in range(2):
        chunks.append((tplane.at[pl.ds(h * H, H)], out.at[b, CB, pl.ds(h * H, H)]))

    bufs = (bufa, bufb)
    sems = (sema, semb)
    in_flight = [None, None]
    for i, (src, dst) in enumerate(chunks):
        slot = i % 2
        if in_flight[slot] is not None:
            in_flight[slot].wait()
        cp_in = pltpu.make_async_copy(src, bufs[slot], sems[slot])
        cp_in.start()
        cp_in.wait()
        cp_out = pltpu.make_async_copy(bufs[slot], dst, sems[slot])
        cp_out.start()
        in_flight[slot] = cp_out
    for cp in in_flight:
        if cp is not None:
            cp.wait()


def _tc_kernel(b1_ref, b2_ref, t1_ref, t2_ref, o1_ref, o2_ref, o3_ref):
    outs = (o1_ref, o2_ref, o3_ref)
    for k in range(3):
        sel = _SELS[k]
        for c in range(CB):
            src = b2_ref if sel[c] >= CB else b1_ref
            outs[k][:, c] = src[:, c]
        t_ref = t2_ref if _THETA_SEL[k] else t1_ref
        tcol = t_ref[:, :]
        outs[k][:, CB] = jnp.broadcast_to(tcol[None, :, :], (BB, IMG, IMG))


def kernel(beta1, beta2, theta1, theta2):
    batch = beta1.shape[0]
    t1_col = jnp.tile(theta1[:, 0], IMG // batch).reshape(IMG, 1)
    t2_col = jnp.tile(theta2[:, 0], IMG // batch).reshape(IMG, 1)
    t4_plane = jnp.broadcast_to(
        t2_col if _THETA_SEL[3] else t1_col, (IMG, IMG)
    )

    out4 = _sc_out4(beta1, beta2, t4_plane)

    beta_spec = pl.BlockSpec((BB, CB, IMG, IMG), lambda b: (b, 0, 0, 0))
    theta_spec = pl.BlockSpec((IMG, 1), lambda b: (0, 0))
    out_spec = pl.BlockSpec((BB, CB + 1, IMG, IMG), lambda b: (b, 0, 0, 0))
    out_shape = jax.ShapeDtypeStruct((batch, CB + 1, IMG, IMG), jnp.float32)

    outs = pl.pallas_call(
        _tc_kernel,
        grid=(batch // BB,),
        in_specs=[beta_spec, beta_spec, theta_spec, theta_spec],
        out_specs=[out_spec] * 3,
        out_shape=[out_shape] * 3,
    )(beta1, beta2, t1_col, t2_col)

    return (*outs, out4, jnp.array(_THETA_SEL, dtype=jnp.int32))


# speedup vs baseline: 58.8305x; 1.3842x over previous
import numpy as np
import jax
import jax.numpy as jnp
from jax.experimental import pallas as pl
from jax.experimental.pallas import tpu as pltpu

CB = 7
IMG = 256


def _randomize_betas_const(cb):
    rnd_lst = np.random.choice(2, cb, p=[0.5, 0.5]).tolist()
    for i in range(len(rnd_lst)):
        ln = len(rnd_lst) - 1
        if rnd_lst[i] == 0:
            rnd_lst[i] = rnd_lst[i] + i
        else:
            rnd_lst[i] = rnd_lst[i] + ln + i
    return rnd_lst


np.random.seed(0)
_SELS = [_randomize_betas_const(CB) for _ in range(4)]
_THETA_SEL = np.random.choice(2, 4, p=[0.5, 0.5]).tolist()
_BITS_PACKED = [
    sum((1 << c) for c in range(CB) if sel[c] >= CB) for sel in _SELS
]


BB = 2


def _recon_kernel(b1_ref, b2_ref, t1_ref, t2_ref, o1_ref, o2_ref, o3_ref, o4_ref):
    outs = (o1_ref, o2_ref, o3_ref, o4_ref)
    for k in range(4):
        sel = _SELS[k]
        for c in range(CB):
            src = b2_ref if sel[c] >= CB else b1_ref
            outs[k][:, c] = src[:, c]
        t_ref = t2_ref if _THETA_SEL[k] else t1_ref
        tcol = t_ref[:, :]
        outs[k][:, CB] = jnp.broadcast_to(tcol[None, :, :], (BB, IMG, IMG))


def kernel(beta1, beta2, theta1, theta2):
    batch = beta1.shape[0]
    t1_col = jnp.tile(theta1[:, 0], IMG // batch).reshape(IMG, 1)
    t2_col = jnp.tile(theta2[:, 0], IMG // batch).reshape(IMG, 1)

    beta_spec = pl.BlockSpec((BB, CB, IMG, IMG), lambda b: (b, 0, 0, 0))
    theta_spec = pl.BlockSpec((IMG, 1), lambda b: (0, 0))
    out_spec = pl.BlockSpec((BB, CB + 1, IMG, IMG), lambda b: (b, 0, 0, 0))
    out_shape = jax.ShapeDtypeStruct((batch, CB + 1, IMG, IMG), jnp.float32)

    outs = pl.pallas_call(
        _recon_kernel,
        grid=(batch // BB,),
        in_specs=[beta_spec, beta_spec, theta_spec, theta_spec],
        out_specs=[out_spec] * 4,
        out_shape=[out_shape] * 4,
    )(beta1, beta2, t1_col, t2_col)

    return (*outs, jnp.array(_THETA_SEL, dtype=jnp.int32))
